# baseline (speedup 1.0000x reference)
import functools

import jax
import jax.numpy as jnp
from jax import lax
from jax.experimental import pallas as pl
from jax.experimental.pallas import tpu as pltpu
from jax.experimental.pallas import tpu_sc as plsc

_N = 10000
_E = 320000
_D = 128
_NCORES = 2
_NSUB = 16
_CH = 125
_NCH = 80
_ZROWS = _N // _NSUB
_NB = 4

_DNUMS = (((1,), (1,)), ((), ()))


def _mm1_body(x_ref, w_ref, o_ref):
    h = lax.dot_general(x_ref[...], w_ref[...], _DNUMS,
                        preferred_element_type=jnp.float32)
    o_ref[...] = h.astype(jnp.bfloat16)


_matmul1 = pl.pallas_call(
    _mm1_body,
    grid=(5,),
    in_specs=[pl.BlockSpec((2000, _D), lambda i: (i, 0)),
              pl.BlockSpec((_D, _D), lambda i: (0, 0))],
    out_specs=pl.BlockSpec((2000, _D), lambda i: (i, 0)),
    out_shape=jax.ShapeDtypeStruct((_N, _D), jnp.bfloat16),
)


def _mm2_body(p_ref, w_ref, o_ref):
    h = p_ref[0].astype(jnp.float32) + p_ref[1].astype(jnp.float32)
    h = jnp.maximum(h, 0.0)
    h = lax.dot_general(h, w_ref[...], _DNUMS,
                        preferred_element_type=jnp.float32)
    o_ref[...] = h.astype(jnp.bfloat16)


_matmul2 = pl.pallas_call(
    _mm2_body,
    grid=(5,),
    in_specs=[pl.BlockSpec((2, 2000, _D), lambda i: (0, i, 0)),
              pl.BlockSpec((_D, _D), lambda i: (0, 0))],
    out_specs=pl.BlockSpec((2000, _D), lambda i: (i, 0)),
    out_shape=jax.ShapeDtypeStruct((_N, _D), jnp.bfloat16),
)


def _lsm_body(p_ref, o_ref):
    h = p_ref[0].astype(jnp.float32) + p_ref[1].astype(jnp.float32)
    m = jnp.max(h, axis=-1, keepdims=True)
    e = jnp.exp(h - m)
    s = jnp.sum(e, axis=-1, keepdims=True)
    o_ref[...] = h - m - jnp.log(s)


_logsoftmax = pl.pallas_call(
    _lsm_body,
    grid=(5,),
    in_specs=[pl.BlockSpec((2, 2000, _D), lambda i: (0, i, 0))],
    out_specs=pl.BlockSpec((2000, _D), lambda i: (i, 0)),
    out_shape=jax.ShapeDtypeStruct((_N, _D), jnp.float32),
)


def _make_agg():
    mesh = plsc.VectorSubcoreMesh(core_axis_name="c", subcore_axis_name="s")

    @functools.partial(
        pl.kernel,
        mesh=mesh,
        compiler_params=pltpu.CompilerParams(use_tc_tiling_on_sc=False),
        out_type=jax.ShapeDtypeStruct((_NCORES, _N, _D), jnp.bfloat16),
        scratch_types=[
            pltpu.VMEM((_NCH, _CH), jnp.int32),
            pltpu.VMEM((_NCH, _CH), jnp.int32),
            pltpu.VMEM((_CH, _D), jnp.bfloat16),
            pltpu.VMEM((_CH, _D), jnp.bfloat16),
            pltpu.VMEM((_CH, _D), jnp.bfloat16),
            pltpu.VMEM((_CH, _D), jnp.bfloat16),
            pltpu.VMEM_SHARED((_N, _D), jnp.bfloat16),
            pltpu.SemaphoreType.DMA,
            pltpu.SemaphoreType.DMA,
            pltpu.SemaphoreType.DMA,
            pltpu.SemaphoreType.DMA,
            pltpu.SemaphoreType.DMA,
            pltpu.SemaphoreType.DMA,
            pltpu.SemaphoreType.DMA,
            pltpu.SemaphoreType.DMA,
        ],
    )
    def agg(h_hbm, e_hbm, zeros_hbm, out_hbm,
            src_v, dst_v, buf0, buf1, buf2, buf3, acc_sh,
            gsem0, gsem1, gsem2, gsem3, ssem0, ssem1, ssem2, ssem3):
        c = lax.axis_index("c")
        s = lax.axis_index("s")
        base = (c * _NSUB + s) * _NCH
        zbase = s * _ZROWS

        pltpu.sync_copy(e_hbm.at[0, pl.ds(base, _NCH)], src_v)
        pltpu.sync_copy(e_hbm.at[1, pl.ds(base, _NCH)], dst_v)
        pltpu.sync_copy(zeros_hbm, acc_sh.at[pl.ds(zbase, _ZROWS)])
        plsc.subcore_barrier()

        bufs = [buf0, buf1, buf2, buf3]
        gsems = [gsem0, gsem1, gsem2, gsem3]
        ssems = [ssem0, ssem1, ssem2, ssem3]

        for b in range(_NB):
            pltpu.async_copy(h_hbm.at[src_v.at[b]], bufs[b], gsems[b])

        @pl.loop(0, _NCH, step=_NB)
        def _edges(i):
            for b in range(_NB):
                cc = i + b
                pltpu.make_async_copy(
                    h_hbm.at[src_v.at[cc]], bufs[b], gsems[b]).wait()
                pltpu.async_copy(
                    bufs[b], acc_sh.at[dst_v.at[cc]], ssems[b], add=True)
                d = (b - 2) % _NB
                cd = cc - 2

                @pl.when(jnp.logical_and(cd >= 0, cd + _NB < _NCH))
                def _():
                    pltpu.make_async_copy(
                        bufs[d], acc_sh.at[dst_v.at[cd]], ssems[d]).wait()
                    pltpu.async_copy(
                        h_hbm.at[src_v.at[cd + _NB]], bufs[d], gsems[d])

        for b in range(_NB):
            pltpu.make_async_copy(
                bufs[b], acc_sh.at[dst_v.at[_NCH - _NB + b]], ssems[b]).wait()

        plsc.subcore_barrier()
        pltpu.sync_copy(acc_sh.at[pl.ds(zbase, _ZROWS)],
                        out_hbm.at[c, pl.ds(zbase, _ZROWS)])

    return agg


_agg = _make_agg()


def kernel(x, edge_index, W1, W2):
    e3 = edge_index.astype(jnp.int32).reshape(2, -1, _CH)
    zeros = jnp.zeros((_ZROWS, _D), jnp.bfloat16)

    h1 = _matmul1(x, W1)
    p = _agg(h1, e3, zeros)
    h2 = _matmul2(p, W2)
    q = _agg(h2, e3, zeros)
    return _logsoftmax(q)


# speedup vs baseline: 1.0514x; 1.0514x over previous
import functools

import jax
import jax.numpy as jnp
from jax import lax
from jax.experimental import pallas as pl
from jax.experimental.pallas import tpu as pltpu
from jax.experimental.pallas import tpu_sc as plsc

_N = 10000
_E = 320000
_D = 128
_NCORES = 2
_NSUB = 16
_CH = 125
_NCH = 80
_ZROWS = _N // _NSUB
_NB = 4

_DNUMS = (((1,), (1,)), ((), ()))


def _mm1_body(x_ref, w_ref, o_ref):
    h = lax.dot_general(x_ref[...], w_ref[...], _DNUMS,
                        preferred_element_type=jnp.float32,
                        precision=lax.Precision.HIGHEST)
    o_ref[...] = h.astype(jnp.bfloat16)


_matmul1 = pl.pallas_call(
    _mm1_body,
    grid=(5,),
    in_specs=[pl.BlockSpec((2000, _D), lambda i: (i, 0)),
              pl.BlockSpec((_D, _D), lambda i: (0, 0))],
    out_specs=pl.BlockSpec((2000, _D), lambda i: (i, 0)),
    out_shape=jax.ShapeDtypeStruct((_N, _D), jnp.bfloat16),
)


def _mm2_body(a_ref, b_ref, w_ref, o_ref):
    h = a_ref[...].astype(jnp.float32) + b_ref[...].astype(jnp.float32)
    h = jnp.maximum(h, 0.0)
    h = lax.dot_general(h, w_ref[...], _DNUMS,
                        preferred_element_type=jnp.float32,
                        precision=lax.Precision.HIGHEST)
    o_ref[...] = h.astype(jnp.bfloat16)


_matmul2 = pl.pallas_call(
    _mm2_body,
    grid=(5,),
    in_specs=[pl.BlockSpec((2000, _D), lambda i: (i, 0)),
              pl.BlockSpec((2000, _D), lambda i: (i, 0)),
              pl.BlockSpec((_D, _D), lambda i: (0, 0))],
    out_specs=pl.BlockSpec((2000, _D), lambda i: (i, 0)),
    out_shape=jax.ShapeDtypeStruct((_N, _D), jnp.bfloat16),
)


def _lsm_body(a_ref, b_ref, o_ref):
    h = a_ref[...].astype(jnp.float32) + b_ref[...].astype(jnp.float32)
    m = jnp.max(h, axis=-1, keepdims=True)
    e = jnp.exp(h - m)
    s = jnp.sum(e, axis=-1, keepdims=True)
    o_ref[...] = h - m - jnp.log(s)


_logsoftmax = pl.pallas_call(
    _lsm_body,
    grid=(5,),
    in_specs=[pl.BlockSpec((2000, _D), lambda i: (i, 0)),
              pl.BlockSpec((2000, _D), lambda i: (i, 0))],
    out_specs=pl.BlockSpec((2000, _D), lambda i: (i, 0)),
    out_shape=jax.ShapeDtypeStruct((_N, _D), jnp.float32),
)


def _make_agg():
    mesh = plsc.VectorSubcoreMesh(core_axis_name="c", subcore_axis_name="s")

    @functools.partial(
        pl.kernel,
        mesh=mesh,
        compiler_params=pltpu.CompilerParams(use_tc_tiling_on_sc=False),
        out_type=(jax.ShapeDtypeStruct((_N, _D), jnp.bfloat16),
                  jax.ShapeDtypeStruct((_N, _D), jnp.bfloat16)),
        scratch_types=[
            pltpu.VMEM((_NCH, _CH), jnp.int32),
            pltpu.VMEM((_NCH, _CH), jnp.int32),
            pltpu.VMEM((_CH, _D), jnp.bfloat16),
            pltpu.VMEM((_CH, _D), jnp.bfloat16),
            pltpu.VMEM((_CH, _D), jnp.bfloat16),
            pltpu.VMEM((_CH, _D), jnp.bfloat16),
            pltpu.VMEM_SHARED((_N, _D), jnp.bfloat16),
            pltpu.SemaphoreType.DMA,
            pltpu.SemaphoreType.DMA,
            pltpu.SemaphoreType.DMA,
            pltpu.SemaphoreType.DMA,
            pltpu.SemaphoreType.DMA,
            pltpu.SemaphoreType.DMA,
            pltpu.SemaphoreType.DMA,
            pltpu.SemaphoreType.DMA,
        ],
    )
    def agg(h_hbm, e_hbm, zeros_hbm, out0_hbm, out1_hbm,
            src_v, dst_v, buf0, buf1, buf2, buf3, acc_sh,
            gsem0, gsem1, gsem2, gsem3, ssem0, ssem1, ssem2, ssem3):
        c = lax.axis_index("c")
        s = lax.axis_index("s")
        base = (c * _NSUB + s) * _NCH
        zbase = s * _ZROWS

        pltpu.sync_copy(e_hbm.at[0, pl.ds(base, _NCH)], src_v)
        pltpu.sync_copy(e_hbm.at[1, pl.ds(base, _NCH)], dst_v)
        pltpu.sync_copy(zeros_hbm, acc_sh.at[pl.ds(zbase, _ZROWS)])
        plsc.subcore_barrier()

        bufs = [buf0, buf1, buf2, buf3]
        gsems = [gsem0, gsem1, gsem2, gsem3]
        ssems = [ssem0, ssem1, ssem2, ssem3]

        for b in range(_NB):
            pltpu.async_copy(h_hbm.at[src_v.at[b]], bufs[b], gsems[b])

        @pl.loop(0, _NCH, step=_NB)
        def _edges(i):
            for b in range(_NB):
                cc = i + b
                pltpu.make_async_copy(
                    h_hbm.at[src_v.at[cc]], bufs[b], gsems[b]).wait()
                pltpu.async_copy(
                    bufs[b], acc_sh.at[dst_v.at[cc]], ssems[b], add=True)
                d = (b - 2) % _NB
                cd = cc - 2

                @pl.when(jnp.logical_and(cd >= 0, cd + _NB < _NCH))
                def _():
                    pltpu.make_async_copy(
                        bufs[d], acc_sh.at[dst_v.at[cd]], ssems[d]).wait()
                    pltpu.async_copy(
                        h_hbm.at[src_v.at[cd + _NB]], bufs[d], gsems[d])

        for b in range(_NB):
            pltpu.make_async_copy(
                bufs[b], acc_sh.at[dst_v.at[_NCH - _NB + b]], ssems[b]).wait()

        plsc.subcore_barrier()

        @pl.when(c == 0)
        def _():
            pltpu.sync_copy(acc_sh.at[pl.ds(zbase, _ZROWS)],
                            out0_hbm.at[pl.ds(zbase, _ZROWS)])

        @pl.when(c == 1)
        def _():
            pltpu.sync_copy(acc_sh.at[pl.ds(zbase, _ZROWS)],
                            out1_hbm.at[pl.ds(zbase, _ZROWS)])

    return agg


_agg = _make_agg()


def kernel(x, edge_index, W1, W2):
    e3 = edge_index.astype(jnp.int32).reshape(2, -1, _CH)
    zeros = jnp.zeros((_ZROWS, _D), jnp.bfloat16)

    h1 = _matmul1(x, W1)
    p0, p1 = _agg(h1, e3, zeros)
    h2 = _matmul2(p0, p1, W2)
    q0, q1 = _agg(h2, e3, zeros)
    return _logsoftmax(q0, q1)


# speedup vs baseline: 1.0678x; 1.0156x over previous
import functools

import jax
import jax.numpy as jnp
from jax import lax
from jax.experimental import pallas as pl
from jax.experimental.pallas import tpu as pltpu
from jax.experimental.pallas import tpu_sc as plsc

_N = 10000
_E = 320000
_D = 128
_NCORES = 2
_NSUB = 16
_CH = 125
_NCH = 80
_ZROWS = _N // _NSUB
_NB = 4

_DNUMS = (((1,), (1,)), ((), ()))


def _mm1_body(x_ref, w_ref, o_ref):
    h = lax.dot_general(x_ref[...], w_ref[...], _DNUMS,
                        preferred_element_type=jnp.float32)
    o_ref[...] = h.astype(jnp.bfloat16)


_matmul1 = pl.pallas_call(
    _mm1_body,
    grid=(5,),
    in_specs=[pl.BlockSpec((2000, _D), lambda i: (i, 0)),
              pl.BlockSpec((_D, _D), lambda i: (0, 0))],
    out_specs=pl.BlockSpec((2000, _D), lambda i: (i, 0)),
    out_shape=jax.ShapeDtypeStruct((_N, _D), jnp.bfloat16),
)


def _mm2_body(a_ref, b_ref, w_ref, o_ref):
    h = a_ref[...].astype(jnp.float32) + b_ref[...].astype(jnp.float32)
    h = jnp.maximum(h, 0.0)
    h = lax.dot_general(h, w_ref[...], _DNUMS,
                        preferred_element_type=jnp.float32)
    o_ref[...] = h.astype(jnp.bfloat16)


_matmul2 = pl.pallas_call(
    _mm2_body,
    grid=(5,),
    in_specs=[pl.BlockSpec((2000, _D), lambda i: (i, 0)),
              pl.BlockSpec((2000, _D), lambda i: (i, 0)),
              pl.BlockSpec((_D, _D), lambda i: (0, 0))],
    out_specs=pl.BlockSpec((2000, _D), lambda i: (i, 0)),
    out_shape=jax.ShapeDtypeStruct((_N, _D), jnp.bfloat16),
)


def _lsm_body(a_ref, b_ref, o_ref):
    h = a_ref[...].astype(jnp.float32) + b_ref[...].astype(jnp.float32)
    m = jnp.max(h, axis=-1, keepdims=True)
    e = jnp.exp(h - m)
    s = jnp.sum(e, axis=-1, keepdims=True)
    o_ref[...] = h - m - jnp.log(s)


_logsoftmax = pl.pallas_call(
    _lsm_body,
    grid=(5,),
    in_specs=[pl.BlockSpec((2000, _D), lambda i: (i, 0)),
              pl.BlockSpec((2000, _D), lambda i: (i, 0))],
    out_specs=pl.BlockSpec((2000, _D), lambda i: (i, 0)),
    out_shape=jax.ShapeDtypeStruct((_N, _D), jnp.float32),
)


def _make_agg():
    mesh = plsc.VectorSubcoreMesh(core_axis_name="c", subcore_axis_name="s")

    @functools.partial(
        pl.kernel,
        mesh=mesh,
        compiler_params=pltpu.CompilerParams(use_tc_tiling_on_sc=False),
        out_type=(jax.ShapeDtypeStruct((_N, _D), jnp.bfloat16),
                  jax.ShapeDtypeStruct((_N, _D), jnp.bfloat16)),
        scratch_types=[
            pltpu.VMEM((_NCH, _CH), jnp.int32),
            pltpu.VMEM((_NCH, _CH), jnp.int32),
            pltpu.VMEM((_CH, _D), jnp.bfloat16),
            pltpu.VMEM((_CH, _D), jnp.bfloat16),
            pltpu.VMEM((_CH, _D), jnp.bfloat16),
            pltpu.VMEM((_CH, _D), jnp.bfloat16),
            pltpu.VMEM_SHARED((_N, _D), jnp.bfloat16),
            pltpu.SemaphoreType.DMA,
            pltpu.SemaphoreType.DMA,
            pltpu.SemaphoreType.DMA,
            pltpu.SemaphoreType.DMA,
            pltpu.SemaphoreType.DMA,
            pltpu.SemaphoreType.DMA,
            pltpu.SemaphoreType.DMA,
            pltpu.SemaphoreType.DMA,
        ],
    )
    def agg(h_hbm, e_hbm, zeros_hbm, out0_hbm, out1_hbm,
            src_v, dst_v, buf0, buf1, buf2, buf3, acc_sh,
            gsem0, gsem1, gsem2, gsem3, ssem0, ssem1, ssem2, ssem3):
        c = lax.axis_index("c")
        s = lax.axis_index("s")
        base = (c * _NSUB + s) * _NCH
        zbase = s * _ZROWS

        pltpu.sync_copy(e_hbm.at[0, pl.ds(base, _NCH)], src_v)
        pltpu.sync_copy(e_hbm.at[1, pl.ds(base, _NCH)], dst_v)
        pltpu.sync_copy(zeros_hbm, acc_sh.at[pl.ds(zbase, _ZROWS)])
        plsc.subcore_barrier()

        bufs = [buf0, buf1, buf2, buf3]
        gsems = [gsem0, gsem1, gsem2, gsem3]
        ssems = [ssem0, ssem1, ssem2, ssem3]

        for b in range(_NB):
            pltpu.async_copy(h_hbm.at[src_v.at[b]], bufs[b], gsems[b])

        @pl.loop(0, _NCH, step=_NB)
        def _edges(i):
            for b in range(_NB):
                cc = i + b
                pltpu.make_async_copy(
                    h_hbm.at[src_v.at[cc]], bufs[b], gsems[b]).wait()
                pltpu.async_copy(
                    bufs[b], acc_sh.at[dst_v.at[cc]], ssems[b], add=True)
                d = (b - 2) % _NB
                cd = cc - 2

                @pl.when(jnp.logical_and(cd >= 0, cd + _NB < _NCH))
                def _():
                    pltpu.make_async_copy(
                        bufs[d], acc_sh.at[dst_v.at[cd]], ssems[d]).wait()
                    pltpu.async_copy(
                        h_hbm.at[src_v.at[cd + _NB]], bufs[d], gsems[d])

        for b in range(_NB):
            pltpu.make_async_copy(
                bufs[b], acc_sh.at[dst_v.at[_NCH - _NB + b]], ssems[b]).wait()

        plsc.subcore_barrier()

        @pl.when(c == 0)
        def _():
            pltpu.sync_copy(acc_sh.at[pl.ds(zbase, _ZROWS)],
                            out0_hbm.at[pl.ds(zbase, _ZROWS)])

        @pl.when(c == 1)
        def _():
            pltpu.sync_copy(acc_sh.at[pl.ds(zbase, _ZROWS)],
                            out1_hbm.at[pl.ds(zbase, _ZROWS)])

    return agg


_agg = _make_agg()


def kernel(x, edge_index, W1, W2):
    e3 = edge_index.astype(jnp.int32).reshape(2, -1, _CH)
    zeros = jnp.zeros((_ZROWS, _D), jnp.bfloat16)

    h1 = _matmul1(x, W1)
    p0, p1 = _agg(h1, e3, zeros)
    h2 = _matmul2(p0, p1, W2)
    q0, q1 = _agg(h2, e3, zeros)
    return _logsoftmax(q0, q1)
